# baseline (speedup 1.0000x reference)
import jax
import jax.numpy as jnp
from jax import lax
from jax.experimental import pallas as pl
from jax.experimental.pallas import tpu as pltpu
from jax.experimental.pallas import tpu_sc as plsc

_B = 16384
_L = 200
_EMBED = 16
_TOTAL = _B * _L
_NC = 2
_NS = 16
_NW = _NC * _NS
_UNIT = 2048
_NUNITS = _TOTAL // _UNIT
_PER_W = _NUNITS // _NW
_TCP = 64
_LSLAB = _NC * 128 * 8 * 128
_E8SLAB = 128 * 8 * 128


def _build():
    mesh = plsc.VectorSubcoreMesh(core_axis_name="c", subcore_axis_name="s")

    @pl.kernel(
        out_type=jax.ShapeDtypeStruct((_TOTAL * _EMBED,), jnp.float32),
        mesh=mesh,
        scratch_types=[
            pltpu.VMEM((2, _UNIT), jnp.int32),
            pltpu.VMEM((2, _UNIT, _EMBED), jnp.float32),
            pltpu.VMEM((8 * 2 * _UNIT,), jnp.float32),
            pltpu.SemaphoreType.DMA,
            pltpu.SemaphoreType.DMA,
            pltpu.SemaphoreType.DMA,
        ],
        compiler_params=pltpu.CompilerParams(
            use_tc_tiling_on_sc=False, needs_layout_passes=False
        ),
    )
    def emb_gather(idx_hbm, table_hbm, out_hbm, idx_v, rows_v, trans_v, gsem0, gsem1, osem):
        wid = lax.axis_index("s") * _NC + lax.axis_index("c")
        g0 = wid * _PER_W
        gsems = [gsem0, gsem1]
        iota16 = lax.iota(jnp.int32, 16)
        cols = [(iota16 + d) & 15 for d in range(16)]
        eoffs = [((c >> 3) << 11) + ((c & 7) << 7) for c in cols]

        def fire(g, b):
            tr = g // _TCP
            tc0 = (g % _TCP) * 2
            off = tr * (128 * 8 * 128) + tc0 * 1024
            pltpu.sync_copy(idx_hbm.at[pl.ds(off, _UNIT)], idx_v.at[b])
            pltpu.async_copy(table_hbm.at[idx_v.at[b]], rows_v.at[b], gsems[b])

        def wait_writes():
            for _ in range(16):
                pltpu.make_async_copy(
                    trans_v.at[pl.ds(0, _UNIT)], out_hbm.at[pl.ds(0, _UNIT)], osem
                ).wait()

        def process(g, b):
            pltpu.make_async_copy(
                table_hbm.at[idx_v.at[b]], rows_v.at[b], gsems[b]
            ).wait()
            tr = g // _TCP
            tc0 = (g % _TCP) * 2
            l0 = tr * 8

            @pl.loop(0, 8)
            def _s(s):
                for tcp in range(2):

                    @pl.loop(0, 8)
                    def _lb(lb):
                        rbase = tcp * 1024 + s * 128 + lb * 16 + iota16
                        wbase = s * 4096 + tcp * 1024 + lb * 16 + iota16
                        for d in range(16):
                            vec = plsc.load_gather(rows_v.at[b], [rbase, cols[d]])
                            plsc.store_scatter(trans_v, [wbase + eoffs[d]], vec)

                for e8 in range(2):
                    q = (l0 + s) * _LSLAB + e8 * _E8SLAB + tc0 * 1024
                    pltpu.async_copy(
                        trans_v.at[pl.ds(s * 4096 + e8 * 2048, _UNIT)],
                        out_hbm.at[pl.ds(q, _UNIT)],
                        osem,
                    )

        fire(g0, 0)

        @pl.loop(0, _PER_W, step=2)
        def _unit(k):
            fire(g0 + k + 1, 1)

            @pl.when(k > 0)
            def _():
                wait_writes()

            process(g0 + k, 0)

            @pl.when(k + 2 < _PER_W)
            def _():
                fire(g0 + k + 2, 0)

            wait_writes()
            process(g0 + k + 1, 1)

        wait_writes()

    return emb_gather


_emb_gather = _build()

_ROWS = 1000001
_TCH = 2032
_NFULL = _ROWS // _TCH
_TAIL = _ROWS - _NFULL * _TCH
_W = _TCH + 8


def _build_transpose():
    mesh = plsc.VectorSubcoreMesh(core_axis_name="c", subcore_axis_name="s")

    @pl.kernel(
        out_type=jax.ShapeDtypeStruct((_ROWS, _EMBED), jnp.float32),
        mesh=mesh,
        scratch_types=[
            pltpu.VMEM((2, _EMBED * _W), jnp.float32),
            pltpu.VMEM((2, _TCH, _EMBED), jnp.float32),
            pltpu.SemaphoreType.DMA,
            pltpu.SemaphoreType.DMA,
            pltpu.SemaphoreType.DMA,
            pltpu.SemaphoreType.DMA,
        ],
        compiler_params=pltpu.CompilerParams(
            use_tc_tiling_on_sc=False, needs_layout_passes=False
        ),
    )
    def table_transpose(tfeat_hbm, rows_hbm, svmem, tvmem, ssem0, ssem1, wsem0, wsem1):
        wid = lax.axis_index("s") * _NC + lax.axis_index("c")
        iota16 = lax.iota(jnp.int32, 16)
        cols = [(iota16 + d) & 15 for d in range(16)]
        voffs = [c * _W + (c & 7) for c in cols]
        ssems = [ssem0, ssem1]
        wsems = [wsem0, wsem1]

        def fire(c, b, n, rl):
            i0 = c * _TCH
            for e in range(16):
                a8 = e * _ROWS + i0 - (e % 8)
                pltpu.async_copy(
                    tfeat_hbm.at[pl.ds(a8, rl)],
                    svmem.at[b, pl.ds(e * _W, rl)],
                    ssems[b],
                )

        def transpose(c, b, n, rl):
            for e in range(16):
                pltpu.make_async_copy(
                    tfeat_hbm.at[pl.ds(0, rl)],
                    svmem.at[b, pl.ds(0, rl)],
                    ssems[b],
                ).wait()

            @pl.loop(0, (n + 15) // 16)
            def _kb(kb):
                k0 = kb * 16 + iota16
                for d in range(16):
                    vec = plsc.load_gather(svmem.at[b], [voffs[d] + k0])
                    plsc.store_scatter(tvmem.at[b], [k0, cols[d]], vec)

            pltpu.async_copy(
                tvmem.at[b, pl.ds(0, n)],
                rows_hbm.at[pl.ds(c * _TCH, n)],
                wsems[b],
            )

        def drain_write(b, n):
            pltpu.make_async_copy(
                tvmem.at[b, pl.ds(0, n)], rows_hbm.at[pl.ds(0, n)], wsems[b]
            ).wait()

        fire(wid, 0, _TCH, _TCH + 8)

        @pl.loop(0, 16, step=2)
        def _pair(k):
            c0 = wid + k * _NW
            c1 = c0 + _NW

            @pl.when(c1 < _NFULL)
            def _():
                fire(c1, 1, _TCH, _TCH + 8)

            @pl.when(c0 >= 2 * _NW)
            def _():
                drain_write(0, _TCH)

            transpose(c0, 0, _TCH, _TCH + 8)

            @pl.when(c0 + 2 * _NW < _NFULL)
            def _():
                fire(c0 + 2 * _NW, 0, _TCH, _TCH + 8)

            @pl.when(c1 < _NFULL)
            def _():
                @pl.when(c1 >= 3 * _NW)
                def _():
                    drain_write(1, _TCH)

                transpose(c1, 1, _TCH, _TCH + 8)

        @pl.when(wid == _NW - 1)
        def _():
            drain_write(0, _TCH)
            fire(_NFULL, 0, _TAIL, _TAIL + 7)
            transpose(_NFULL, 0, _TAIL, _TAIL + 7)
            drain_write(1, _TCH)
            drain_write(0, _TAIL)

        @pl.when(wid != _NW - 1)
        def _():
            drain_write(0, _TCH)
            drain_write(1, _TCH)

    return table_transpose


_table_transpose = _build_transpose()


@jax.jit
def kernel(log_seqs, item_emb_weight):
    idx4 = log_seqs.reshape(128, 128, 25, 8)
    idxp = jnp.transpose(idx4, (2, 0, 3, 1)).reshape(_TOTAL)
    rows = _table_transpose(jnp.transpose(item_emb_weight).reshape(_EMBED * _ROWS))
    out = _emb_gather(idxp, rows)
    out5 = out.reshape(200, 2, 128, 8, 128)
    return jnp.transpose(out5, (2, 4, 0, 1, 3)).reshape(_B, _L, _EMBED)


# speedup vs baseline: 2.1349x; 2.1349x over previous
import jax
import jax.numpy as jnp
from jax import lax
from jax.experimental import pallas as pl
from jax.experimental.pallas import tpu as pltpu
from jax.experimental.pallas import tpu_sc as plsc

_B = 16384
_L = 200
_EMBED = 16
_TOTAL = _B * _L
_NC = 2
_NS = 16
_NW = _NC * _NS
_UNIT = 2048
_NUNITS = _TOTAL // _UNIT
_PER_W = _NUNITS // _NW
_TCP = 64
_LSLAB = _NC * 128 * 8 * 128
_E8SLAB = 128 * 8 * 128


def _build():
    mesh = plsc.VectorSubcoreMesh(core_axis_name="c", subcore_axis_name="s")

    @pl.kernel(
        out_type=jax.ShapeDtypeStruct((_TOTAL * _EMBED,), jnp.float32),
        mesh=mesh,
        scratch_types=[
            pltpu.VMEM((2, _UNIT), jnp.int32),
            pltpu.VMEM((2, _UNIT, _EMBED), jnp.float32),
            pltpu.VMEM((8 * 2 * _UNIT,), jnp.float32),
            pltpu.SemaphoreType.DMA,
            pltpu.SemaphoreType.DMA,
            pltpu.SemaphoreType.DMA,
            pltpu.SemaphoreType.DMA,
            pltpu.SemaphoreType.DMA,
        ],
        compiler_params=pltpu.CompilerParams(
            use_tc_tiling_on_sc=False, needs_layout_passes=False
        ),
    )
    def emb_gather(
        idx_hbm, table_hbm, out_hbm, idx_v, rows_v, trans_v,
        gsem0, gsem1, isem0, isem1, osem,
    ):
        wid = lax.axis_index("s") * _NC + lax.axis_index("c")
        g0 = wid * _PER_W
        gend = g0 + _PER_W
        gsems = [gsem0, gsem1]
        isems = [isem0, isem1]
        iota16 = lax.iota(jnp.int32, 16)
        cols = [(iota16 + d) & 15 for d in range(16)]
        eoffs = [((c >> 3) << 11) + ((c & 7) << 7) for c in cols]

        def idx_off(g):
            tr = g // _TCP
            tc0 = (g % _TCP) * 2
            return tr * (128 * 8 * 128) + tc0 * 1024

        def prefetch_idx(g, b):
            pltpu.async_copy(
                idx_hbm.at[pl.ds(idx_off(g), _UNIT)], idx_v.at[b], isems[b]
            )

        def fire(g, b):
            pltpu.make_async_copy(
                idx_hbm.at[pl.ds(0, _UNIT)], idx_v.at[b], isems[b]
            ).wait()
            pltpu.async_copy(table_hbm.at[idx_v.at[b]], rows_v.at[b], gsems[b])

        def wait_writes():
            for _ in range(16):
                pltpu.make_async_copy(
                    trans_v.at[pl.ds(0, _UNIT)], out_hbm.at[pl.ds(0, _UNIT)], osem
                ).wait()

        def process(g, b):
            pltpu.make_async_copy(
                table_hbm.at[idx_v.at[b]], rows_v.at[b], gsems[b]
            ).wait()

            @pl.when(g + 2 < gend)
            def _():
                prefetch_idx(g + 2, b)

            tr = g // _TCP
            tc0 = (g % _TCP) * 2
            l0 = tr * 8

            @pl.loop(0, 8)
            def _s(s):
                for tcp in range(2):

                    @pl.loop(0, 8)
                    def _lb(lb):
                        rbase = tcp * 1024 + s * 128 + lb * 16 + iota16
                        wbase = s * 4096 + tcp * 1024 + lb * 16 + iota16
                        for d in range(16):
                            vec = plsc.load_gather(rows_v.at[b], [rbase, cols[d]])
                            plsc.store_scatter(trans_v, [wbase + eoffs[d]], vec)

                for e8 in range(2):
                    q = (l0 + s) * _LSLAB + e8 * _E8SLAB + tc0 * 1024
                    pltpu.async_copy(
                        trans_v.at[pl.ds(s * 4096 + e8 * 2048, _UNIT)],
                        out_hbm.at[pl.ds(q, _UNIT)],
                        osem,
                    )

        prefetch_idx(g0, 0)
        fire(g0, 0)
        prefetch_idx(g0 + 1, 1)

        @pl.loop(0, _PER_W, step=2)
        def _unit(k):
            fire(g0 + k + 1, 1)

            @pl.when(k > 0)
            def _():
                wait_writes()

            process(g0 + k, 0)

            @pl.when(k + 2 < _PER_W)
            def _():
                fire(g0 + k + 2, 0)

            wait_writes()
            process(g0 + k + 1, 1)

        wait_writes()

    return emb_gather


_emb_gather = _build()


@jax.jit
def kernel(log_seqs, item_emb_weight):
    idx4 = log_seqs.reshape(128, 128, 25, 8)
    idxp = jnp.transpose(idx4, (2, 0, 3, 1)).reshape(_TOTAL)
    out = _emb_gather(idxp, item_emb_weight)
    out5 = out.reshape(200, 2, 128, 8, 128)
    return jnp.transpose(out5, (2, 4, 0, 1, 3)).reshape(_B, _L, _EMBED)


# speedup vs baseline: 2.1355x; 1.0003x over previous
import jax
import jax.numpy as jnp
from jax import lax
from jax.experimental import pallas as pl
from jax.experimental.pallas import tpu as pltpu
from jax.experimental.pallas import tpu_sc as plsc

_B = 16384
_L = 200
_EMBED = 16
_TOTAL = _B * _L
_NC = 2
_NS = 16
_NW = _NC * _NS
_UNIT = 2048
_NUNITS = _TOTAL // _UNIT
_PER_W = _NUNITS // _NW
_TCP = 64
_LSLAB = _NC * 128 * 8 * 128
_E8SLAB = 128 * 8 * 128


def _build():
    mesh = plsc.VectorSubcoreMesh(core_axis_name="c", subcore_axis_name="s")

    @pl.kernel(
        out_type=jax.ShapeDtypeStruct((_TOTAL * _EMBED,), jnp.float32),
        mesh=mesh,
        scratch_types=[
            pltpu.VMEM((2, _UNIT), jnp.int32),
            pltpu.VMEM((2, _UNIT, _EMBED), jnp.float32),
            pltpu.VMEM((8 * 2 * _UNIT,), jnp.float32),
            pltpu.SemaphoreType.DMA,
            pltpu.SemaphoreType.DMA,
            pltpu.SemaphoreType.DMA,
            pltpu.SemaphoreType.DMA,
            pltpu.SemaphoreType.DMA,
        ],
        compiler_params=pltpu.CompilerParams(
            use_tc_tiling_on_sc=False, needs_layout_passes=False
        ),
    )
    def emb_gather(
        idx_hbm, table_hbm, out_hbm, idx_v, rows_v, trans_v,
        gsem0, gsem1, isem0, isem1, osem,
    ):
        wid = lax.axis_index("s") * _NC + lax.axis_index("c")
        g0 = wid * _PER_W
        gend = g0 + _PER_W
        gsems = [gsem0, gsem1]
        isems = [isem0, isem1]
        iota16 = lax.iota(jnp.int32, 16)
        cols = [(iota16 + d) & 15 for d in range(16)]
        eoffs = [((c >> 3) << 11) + ((c & 7) << 7) for c in cols]

        def idx_off(g):
            tr = g // _TCP
            tc0 = (g % _TCP) * 2
            return tr * (128 * 8 * 128) + tc0 * 1024

        def prefetch_idx(g, b):
            pltpu.async_copy(
                idx_hbm.at[pl.ds(idx_off(g), _UNIT)], idx_v.at[b], isems[b]
            )

        def fire(g, b):
            pltpu.make_async_copy(
                idx_hbm.at[pl.ds(0, _UNIT)], idx_v.at[b], isems[b]
            ).wait()
            h = _UNIT // 2
            pltpu.async_copy(
                table_hbm.at[idx_v.at[b, pl.ds(0, h)]],
                rows_v.at[b, pl.ds(0, h)],
                gsems[b],
            )
            pltpu.async_copy(
                table_hbm.at[idx_v.at[b, pl.ds(h, h)]],
                rows_v.at[b, pl.ds(h, h)],
                gsems[b],
            )

        def wait_writes():
            for _ in range(16):
                pltpu.make_async_copy(
                    trans_v.at[pl.ds(0, _UNIT)], out_hbm.at[pl.ds(0, _UNIT)], osem
                ).wait()

        def process(g, b):
            pltpu.make_async_copy(
                table_hbm.at[idx_v.at[b]], rows_v.at[b], gsems[b]
            ).wait()

            @pl.when(g + 2 < gend)
            def _():
                prefetch_idx(g + 2, b)

            tr = g // _TCP
            tc0 = (g % _TCP) * 2
            l0 = tr * 8

            @pl.loop(0, 8)
            def _s(s):
                for tcp in range(2):

                    @pl.loop(0, 8)
                    def _lb(lb):
                        rbase = tcp * 1024 + s * 128 + lb * 16 + iota16
                        wbase = s * 4096 + tcp * 1024 + lb * 16 + iota16
                        for d in range(16):
                            vec = plsc.load_gather(rows_v.at[b], [rbase, cols[d]])
                            plsc.store_scatter(trans_v, [wbase + eoffs[d]], vec)

                for e8 in range(2):
                    q = (l0 + s) * _LSLAB + e8 * _E8SLAB + tc0 * 1024
                    pltpu.async_copy(
                        trans_v.at[pl.ds(s * 4096 + e8 * 2048, _UNIT)],
                        out_hbm.at[pl.ds(q, _UNIT)],
                        osem,
                    )

        prefetch_idx(g0, 0)
        fire(g0, 0)
        prefetch_idx(g0 + 1, 1)

        @pl.loop(0, _PER_W, step=2)
        def _unit(k):
            fire(g0 + k + 1, 1)

            @pl.when(k > 0)
            def _():
                wait_writes()

            process(g0 + k, 0)

            @pl.when(k + 2 < _PER_W)
            def _():
                fire(g0 + k + 2, 0)

            wait_writes()
            process(g0 + k + 1, 1)

        wait_writes()

    return emb_gather


_emb_gather = _build()


@jax.jit
def kernel(log_seqs, item_emb_weight):
    idx4 = log_seqs.reshape(128, 128, 25, 8)
    idxp = jnp.transpose(idx4, (2, 0, 3, 1)).reshape(_TOTAL)
    out = _emb_gather(idxp, item_emb_weight)
    out5 = out.reshape(200, 2, 128, 8, 128)
    return jnp.transpose(out5, (2, 4, 0, 1, 3)).reshape(_B, _L, _EMBED)


# speedup vs baseline: 2.9271x; 1.3707x over previous
import jax
import jax.numpy as jnp
from jax import lax
from jax.experimental import pallas as pl
from jax.experimental.pallas import tpu as pltpu
from jax.experimental.pallas import tpu_sc as plsc

_B = 16384
_L = 200
_EMBED = 16
_TOTAL = _B * _L
_NC = 2
_NS = 16
_NW = _NC * _NS
_UNIT = 2048
_NUNITS = _TOTAL // _UNIT
_PER_W = _NUNITS // _NW
_TCP = 64
_LSLAB = _NC * 128 * 8 * 128
_E8SLAB = 128 * 8 * 128


def _build():
    mesh = plsc.VectorSubcoreMesh(core_axis_name="c", subcore_axis_name="s")

    @pl.kernel(
        out_type=jax.ShapeDtypeStruct((_TOTAL * _EMBED,), jnp.float32),
        mesh=mesh,
        scratch_types=[
            pltpu.VMEM((2, _UNIT), jnp.int32),
            pltpu.VMEM((2, _UNIT, _EMBED), jnp.float32),
            pltpu.VMEM((8 * 2 * _UNIT,), jnp.float32),
            pltpu.SemaphoreType.DMA,
            pltpu.SemaphoreType.DMA,
            pltpu.SemaphoreType.DMA,
            pltpu.SemaphoreType.DMA,
            pltpu.SemaphoreType.DMA,
        ],
        compiler_params=pltpu.CompilerParams(
            use_tc_tiling_on_sc=False, needs_layout_passes=False
        ),
    )
    def emb_gather(
        idx_hbm, table_hbm, out_hbm, idx_v, rows_v, trans_v,
        gsem0, gsem1, isem0, isem1, osem,
    ):
        wid = lax.axis_index("s") * _NC + lax.axis_index("c")
        g0 = wid * _PER_W
        gend = g0 + _PER_W
        gsems = [gsem0, gsem1]
        isems = [isem0, isem1]
        iota16 = lax.iota(jnp.int32, 16)
        cols = [(iota16 + d) & 15 for d in range(16)]
        eoffs = [((c >> 3) << 11) + ((c & 7) << 7) for c in cols]

        def idx_off(g):
            tr = g // _TCP
            tc0 = (g % _TCP) * 2
            return tr * (128 * 8 * 128) + tc0 * 1024

        def prefetch_idx(g, b):
            pltpu.async_copy(
                idx_hbm.at[pl.ds(idx_off(g), _UNIT)], idx_v.at[b], isems[b]
            )

        def fire(g, b):
            pltpu.make_async_copy(
                idx_hbm.at[pl.ds(0, _UNIT)], idx_v.at[b], isems[b]
            ).wait()
            h = _UNIT // 2
            pltpu.async_copy(
                table_hbm.at[idx_v.at[b, pl.ds(0, h)]],
                rows_v.at[b, pl.ds(0, h)],
                gsems[b],
            )
            pltpu.async_copy(
                table_hbm.at[idx_v.at[b, pl.ds(h, h)]],
                rows_v.at[b, pl.ds(h, h)],
                gsems[b],
            )

        def wait_writes():
            for _ in range(16):
                pltpu.make_async_copy(
                    trans_v.at[pl.ds(0, _UNIT)], out_hbm.at[pl.ds(0, _UNIT)], osem
                ).wait()

        def process(g, b):
            pltpu.make_async_copy(
                table_hbm.at[idx_v.at[b]], rows_v.at[b], gsems[b]
            ).wait()

            @pl.when(g + 2 < gend)
            def _():
                prefetch_idx(g + 2, b)

            tr = g // _TCP
            tc0 = (g % _TCP) * 2
            l0 = tr * 8

            @pl.loop(0, 8)
            def _s(s):
                for tcp in range(2):

                    @plsc.parallel_loop(0, 8)
                    def _lb(lb):
                        rbase = tcp * 1024 + s * 128 + lb * 16 + iota16
                        wbase = s * 4096 + tcp * 1024 + lb * 16 + iota16
                        for d in range(16):
                            vec = plsc.load_gather(rows_v.at[b], [rbase, cols[d]])
                            plsc.store_scatter(trans_v, [wbase + eoffs[d]], vec)

                for e8 in range(2):
                    q = (l0 + s) * _LSLAB + e8 * _E8SLAB + tc0 * 1024
                    pltpu.async_copy(
                        trans_v.at[pl.ds(s * 4096 + e8 * 2048, _UNIT)],
                        out_hbm.at[pl.ds(q, _UNIT)],
                        osem,
                    )

        prefetch_idx(g0, 0)
        fire(g0, 0)
        prefetch_idx(g0 + 1, 1)

        @pl.loop(0, _PER_W, step=2)
        def _unit(k):
            fire(g0 + k + 1, 1)

            @pl.when(k > 0)
            def _():
                wait_writes()

            process(g0 + k, 0)

            @pl.when(k + 2 < _PER_W)
            def _():
                fire(g0 + k + 2, 0)

            wait_writes()
            process(g0 + k + 1, 1)

        wait_writes()

    return emb_gather


_emb_gather = _build()


@jax.jit
def kernel(log_seqs, item_emb_weight):
    idx4 = log_seqs.reshape(128, 128, 25, 8)
    idxp = jnp.transpose(idx4, (2, 0, 3, 1)).reshape(_TOTAL)
    out = _emb_gather(idxp, item_emb_weight)
    out5 = out.reshape(200, 2, 128, 8, 128)
    return jnp.transpose(out5, (2, 4, 0, 1, 3)).reshape(_B, _L, _EMBED)


# speedup vs baseline: 6.4197x; 2.1932x over previous
import jax
import jax.numpy as jnp
from jax import lax
from jax.experimental import pallas as pl
from jax.experimental.pallas import tpu as pltpu
from jax.experimental.pallas import tpu_sc as plsc

_B = 16384
_L = 200
_EMBED = 16
_TOTAL = _B * _L
_NC = 2
_NS = 16
_NW = _NC * _NS
_UNIT = 2048
_NUNITS = _TOTAL // _UNIT
_PER_W = _NUNITS // _NW
_TCP = 64
_LSLAB = _NC * 128 * 8 * 128
_E8SLAB = 128 * 8 * 128


def _build():
    mesh = plsc.VectorSubcoreMesh(core_axis_name="c", subcore_axis_name="s")

    @pl.kernel(
        out_type=jax.ShapeDtypeStruct((_TOTAL * _EMBED,), jnp.float32),
        mesh=mesh,
        scratch_types=[
            pltpu.VMEM((2, _UNIT), jnp.int32),
            pltpu.VMEM((2, _UNIT, _EMBED), jnp.float32),
            pltpu.VMEM((8 * 2 * _UNIT,), jnp.float32),
            pltpu.SemaphoreType.DMA,
            pltpu.SemaphoreType.DMA,
            pltpu.SemaphoreType.DMA,
            pltpu.SemaphoreType.DMA,
            pltpu.SemaphoreType.DMA,
        ],
        compiler_params=pltpu.CompilerParams(
            use_tc_tiling_on_sc=False, needs_layout_passes=False
        ),
    )
    def emb_gather(
        idx_hbm, table_hbm, out_hbm, idx_v, rows_v, trans_v,
        gsem0, gsem1, isem0, isem1, osem,
    ):
        wid = lax.axis_index("s") * _NC + lax.axis_index("c")
        g0 = wid * _PER_W
        gend = g0 + _PER_W
        gsems = [gsem0, gsem1]
        isems = [isem0, isem1]
        iota16 = lax.iota(jnp.int32, 16)
        cols = [(iota16 + d) & 15 for d in range(16)]
        eoffs = [((c >> 3) << 11) + ((c & 7) << 7) for c in cols]

        def idx_off(g):
            tr = g // _TCP
            tc0 = (g % _TCP) * 2
            return tr * (128 * 8 * 128) + tc0 * 1024

        def prefetch_idx(g, b):
            pltpu.async_copy(
                idx_hbm.at[pl.ds(idx_off(g), _UNIT)], idx_v.at[b], isems[b]
            )

        def fire(g, b):
            pltpu.make_async_copy(
                idx_hbm.at[pl.ds(0, _UNIT)], idx_v.at[b], isems[b]
            ).wait()
            h = _UNIT // 2
            pltpu.async_copy(
                table_hbm.at[idx_v.at[b, pl.ds(0, h)]],
                rows_v.at[b, pl.ds(0, h)],
                gsems[b],
            )
            pltpu.async_copy(
                table_hbm.at[idx_v.at[b, pl.ds(h, h)]],
                rows_v.at[b, pl.ds(h, h)],
                gsems[b],
            )

        def wait_writes():
            for _ in range(16):
                pltpu.make_async_copy(
                    trans_v.at[pl.ds(0, _UNIT)], out_hbm.at[pl.ds(0, _UNIT)], osem
                ).wait()

        def process(g, b):
            pltpu.make_async_copy(
                table_hbm.at[idx_v.at[b]], rows_v.at[b], gsems[b]
            ).wait()

            @pl.when(g + 2 < gend)
            def _():
                prefetch_idx(g + 2, b)

            tr = g // _TCP
            tc0 = (g % _TCP) * 2
            l0 = tr * 8

            @pl.loop(0, 8)
            def _s(s):
                for tcp in range(2):

                    @plsc.parallel_loop(0, 8)
                    def _lb(lb):
                        rbase = tcp * 1024 + s * 128 + lb * 16 + iota16
                        wbase = s * 4096 + tcp * 1024 + lb * 16 + iota16
                        for d in range(16):
                            vec = plsc.load_gather(rows_v.at[b], [rbase, cols[d]])
                            plsc.store_scatter(trans_v, [wbase + eoffs[d]], vec)

                for e8 in range(2):
                    q = (l0 + s) * _LSLAB + e8 * _E8SLAB + tc0 * 1024
                    pltpu.async_copy(
                        trans_v.at[pl.ds(s * 4096 + e8 * 2048, _UNIT)],
                        out_hbm.at[pl.ds(q, _UNIT)],
                        osem,
                    )

        prefetch_idx(g0, 0)
        fire(g0, 0)
        prefetch_idx(g0 + 1, 1)

        @pl.loop(0, _PER_W, step=2)
        def _unit(k):
            fire(g0 + k + 1, 1)

            @pl.when(k > 0)
            def _():
                wait_writes()

            process(g0 + k, 0)

            @pl.when(k + 2 < _PER_W)
            def _():
                fire(g0 + k + 2, 0)

            wait_writes()
            process(g0 + k + 1, 1)

        wait_writes()

    return emb_gather


_emb_gather = _build()


_ROWS = 1000001
_R4 = 999936
_RREM = _ROWS - _R4
_TRHALF = 7812 * 1024
_TCH = 1536
_NFC = _R4 // _TCH
_CW = _TCH * 8


def _build_transpose():
    mesh = plsc.VectorSubcoreMesh(core_axis_name="c", subcore_axis_name="s")

    @pl.kernel(
        out_type=jax.ShapeDtypeStruct((_ROWS, _EMBED), jnp.float32),
        mesh=mesh,
        scratch_types=[
            pltpu.VMEM((2, 2 * _CW), jnp.float32),
            pltpu.VMEM((2, _TCH, _EMBED), jnp.float32),
            pltpu.SemaphoreType.DMA,
            pltpu.SemaphoreType.DMA,
            pltpu.SemaphoreType.DMA,
            pltpu.SemaphoreType.DMA,
        ],
        compiler_params=pltpu.CompilerParams(
            use_tc_tiling_on_sc=False, needs_layout_passes=False
        ),
    )
    def table_transpose(t4_hbm, trem_hbm, rows_hbm, svmem, tvmem, ssem0, ssem1, wsem0, wsem1):
        wid = lax.axis_index("s") * _NC + lax.axis_index("c")
        iota16 = lax.iota(jnp.int32, 16)
        cols = [(iota16 + d) & 15 for d in range(16)]
        etoffs = [((c >> 3) * _CW) + ((c & 7) << 7) for c in cols]
        ssems = [ssem0, ssem1]
        wsems = [wsem0, wsem1]

        def fire(c, b):
            for tr in range(2):
                pltpu.async_copy(
                    t4_hbm.at[pl.ds(tr * _TRHALF + c * _CW, _CW)],
                    svmem.at[b, pl.ds(tr * _CW, _CW)],
                    ssems[b],
                )

        def transpose(c, b):
            for tr in range(2):
                pltpu.make_async_copy(
                    t4_hbm.at[pl.ds(0, _CW)],
                    svmem.at[b, pl.ds(0, _CW)],
                    ssems[b],
                ).wait()

            @plsc.parallel_loop(0, 96)
            def _j(j):
                tc = j // 8
                l0 = tc * 1024 + (j % 8) * 16 + iota16
                ivec = tc * 128 + (j % 8) * 16 + iota16
                for d in range(16):
                    vec = plsc.load_gather(svmem.at[b], [etoffs[d] + l0])
                    plsc.store_scatter(tvmem.at[b], [ivec, cols[d]], vec)

            pltpu.async_copy(
                tvmem.at[b], rows_hbm.at[pl.ds(c * _TCH, _TCH)], wsems[b]
            )

        def drain_write(b):
            pltpu.make_async_copy(
                tvmem.at[b], rows_hbm.at[pl.ds(0, _TCH)], wsems[b]
            ).wait()

        fire(wid, 0)

        @pl.loop(0, 22, step=2)
        def _pair(k):
            c0 = wid + k * _NW
            c1 = c0 + _NW

            @pl.when(c1 < _NFC)
            def _():
                fire(c1, 1)

            @pl.when(c0 < _NFC)
            def _():
                @pl.when(c0 >= 2 * _NW)
                def _():
                    drain_write(0)

                transpose(c0, 0)

            @pl.when(c0 + 2 * _NW < _NFC)
            def _():
                fire(c0 + 2 * _NW, 0)

            @pl.when(c1 < _NFC)
            def _():
                @pl.when(c1 >= 3 * _NW)
                def _():
                    drain_write(1)

                transpose(c1, 1)

        drain_write(0)
        drain_write(1)

        @pl.when(wid == _NW - 1)
        def _():
            pltpu.sync_copy(trem_hbm, tvmem.at[0, pl.ds(0, _RREM)])
            pltpu.sync_copy(
                tvmem.at[0, pl.ds(0, _RREM)], rows_hbm.at[pl.ds(_R4, _RREM)]
            )

    return table_transpose


_table_transpose = _build_transpose()


@jax.jit
def kernel(log_seqs, item_emb_weight):
    idx4 = log_seqs.reshape(128, 128, 25, 8)
    idxp = jnp.transpose(idx4, (2, 0, 3, 1)).reshape(_TOTAL)
    t4 = item_emb_weight[:_R4].reshape(7812, 128, 2, 8)
    t4f = jnp.transpose(t4, (2, 0, 3, 1)).reshape(2 * _TRHALF)
    rows = _table_transpose(t4f, item_emb_weight[_R4:])
    out = _emb_gather(idxp, rows)
    out5 = out.reshape(200, 2, 128, 8, 128)
    return jnp.transpose(out5, (2, 4, 0, 1, 3)).reshape(_B, _L, _EMBED)
